# baseline (speedup 1.0000x reference)
import functools
import math

import jax
import jax.numpy as jnp
from jax import lax
from jax.experimental import pallas as pl
from jax.experimental.pallas import tpu as pltpu
from jax.experimental.pallas import tpu_sc as plsc

_L = 16
_NB = 4


def _pos_encoding(T_len, d_model, dtype):
    positions = jnp.arange(T_len, dtype=dtype)[:, None]
    i = jnp.arange(0, d_model, 2, dtype=dtype)
    denominator = jnp.exp(i / d_model * math.log(10000.0))
    pe = jnp.zeros((T_len, d_model), dtype=dtype)
    pe = pe.at[:, 0::2].set(jnp.sin(positions / denominator))
    pe = pe.at[:, 1::2].set(jnp.cos(positions / denominator))
    return pe


def _make_sc_kernel(B, T, V, D, scale):
    try:
        info = plsc.get_sparse_core_info()
        NC, NS = info.num_cores, info.num_subcores
    except ValueError:
        NC, NS = 2, 16
    NW = NC * NS
    assert B % NW == 0 and T % _NB == 0 and D % _L == 0
    C = B // NW
    assert C % _L == 0
    NR = C // _L
    mesh = plsc.VectorSubcoreMesh(
        core_axis_name="c",
        subcore_axis_name="s",
        num_cores=NC,
        num_subcores=NS,
    )

    scratch = [
        pltpu.VMEM((T * D,), jnp.float32),
        pltpu.VMEM((_NB, C), jnp.int32),
        pltpu.VMEM((_NB, C), jnp.int32),
        pltpu.VMEM((_NB, C, 2 * D), jnp.float32),
        pltpu.VMEM((_NB, D, C), jnp.float32),
        [pltpu.SemaphoreType.DMA] * _NB,
        [pltpu.SemaphoreType.DMA] * _NB,
        [pltpu.SemaphoreType.DMA] * _NB,
        [pltpu.SemaphoreType.DMA] * _NB,
    ]

    @functools.partial(
        pl.kernel,
        out_type=jax.ShapeDtypeStruct((T, D, B), jnp.float32),
        mesh=mesh,
        scratch_types=scratch,
        compiler_params=pltpu.CompilerParams(use_tc_tiling_on_sc=True, needs_layout_passes=False),
    )
    def k(xp_hbm, xh_hbm, we2_hbm, pe_hbm, out_hbm,
          pe_v, xp_r, xh_r, rows_r, obuf_r, sem_ip, sem_ih, sem_g, sem_o):
        wid = lax.axis_index("s") * NC + lax.axis_index("c")
        b0 = wid * C
        pltpu.sync_copy(pe_hbm, pe_v)

        def idx_cp(g, b, kind):
            src = (xp_hbm, xh_hbm)[kind]
            dst = (xp_r, xh_r)[kind]
            sem = (sem_ip, sem_ih)[kind]
            return pltpu.make_async_copy(
                src.at[g, pl.ds(b0, C)], dst.at[b], sem[b]
            )

        def gather(g, b):
            return pltpu.make_async_copy(
                we2_hbm.at[xp_r.at[b]], rows_r.at[b], sem_g[b]
            )

        def writeout(g, b):
            return pltpu.make_async_copy(
                obuf_r.at[b], out_hbm.at[g, :, pl.ds(b0, C)], sem_o[b]
            )

        for g in range(3):
            idx_cp(g, g, 0).start()
            idx_cp(g, g, 1).start()
        for g in range(2):
            idx_cp(g, g, 0).wait()
            gather(g, g).start()

        iota = lax.iota(jnp.int32, _L)

        def outer(gg, carry):
            for b in range(_NB):
                g = gg * _NB + b

                @pl.when(g + 3 < T)
                def _():
                    idx_cp(g + 3, (b + 3) % _NB, 0).start()
                    idx_cp(g + 3, (b + 3) % _NB, 1).start()

                @pl.when(g + 2 < T)
                def _():
                    idx_cp(g + 2, (b + 2) % _NB, 0).wait()
                    gather(g + 2, (b + 2) % _NB).start()

                gather(g, b).wait()
                idx_cp(g, b, 1).wait()

                @pl.when(g >= _NB)
                def _():
                    writeout(g - _NB, b).wait()

                rows2d = rows_r.at[b]
                rowv = []
                colv = []
                for r0 in range(NR):
                    hv = xh_r[b, pl.ds(r0 * _L, _L)]
                    rowv.append(iota + (r0 * _L))
                    colv.append(hv * D)
                tD = g * D

                def jloop(j, c, b=b, rows2d=rows2d, rowv=rowv, colv=colv, tD=tD):
                    pe_s = plsc.load_gather(
                        pe_v, [jnp.full((_L,), tD, jnp.int32) + j]
                    )
                    for r0 in range(NR):
                        v = plsc.load_gather(rows2d, [rowv[r0], colv[r0] + j])
                        obuf_r[b, j, pl.ds(r0 * _L, _L)] = v * scale + pe_s
                    return c

                lax.fori_loop(0, D, jloop, 0)
                writeout(g, b).start()
            return carry

        lax.fori_loop(0, T // _NB, outer, 0)
        for g in range(T - _NB, T):
            writeout(g, g % _NB).wait()

    return k


@jax.jit
def kernel(x, We):
    B, T = x.shape
    V, D = We.shape
    scale = math.sqrt(D)
    pe1d = _pos_encoding(T, D, jnp.float32).reshape(T * D)
    xT = x.T
    xp = xT >> 1
    xh = xT & 1
    We2 = We.reshape(V // 2, 2 * D)
    sc = _make_sc_kernel(B, T, V, D, scale)
    out_t = sc(xp, xh, We2, pe1d)
    return out_t.transpose(2, 0, 1)


# speedup vs baseline: 1.2717x; 1.2717x over previous
import functools
import math

import jax
import jax.numpy as jnp
from jax import lax
from jax.experimental import pallas as pl
from jax.experimental.pallas import tpu as pltpu
from jax.experimental.pallas import tpu_sc as plsc

_L = 16
_NB = 4


def _pos_encoding(T_len, d_model, dtype):
    positions = jnp.arange(T_len, dtype=dtype)[:, None]
    i = jnp.arange(0, d_model, 2, dtype=dtype)
    denominator = jnp.exp(i / d_model * math.log(10000.0))
    pe = jnp.zeros((T_len, d_model), dtype=dtype)
    pe = pe.at[:, 0::2].set(jnp.sin(positions / denominator))
    pe = pe.at[:, 1::2].set(jnp.cos(positions / denominator))
    return pe


def _make_sc_kernel(B, T, V, D, scale):
    try:
        info = plsc.get_sparse_core_info()
        NC, NS = info.num_cores, info.num_subcores
    except ValueError:
        NC, NS = 2, 16
    NW = NC * NS
    assert B % NW == 0 and T % _NB == 0 and D % _L == 0
    C = B // NW
    assert C % _L == 0
    NR = C // _L
    mesh = plsc.VectorSubcoreMesh(
        core_axis_name="c",
        subcore_axis_name="s",
        num_cores=NC,
        num_subcores=NS,
    )

    scratch = [
        pltpu.VMEM((T * D,), jnp.float32),
        pltpu.VMEM((_NB, C), jnp.int32),
        pltpu.VMEM((_NB, C), jnp.int32),
        pltpu.VMEM((_NB, C, 2 * D), jnp.float32),
        pltpu.VMEM((_NB, D, C), jnp.float32),
        [pltpu.SemaphoreType.DMA] * _NB,
        [pltpu.SemaphoreType.DMA] * _NB,
        [pltpu.SemaphoreType.DMA] * _NB,
        [pltpu.SemaphoreType.DMA] * _NB,
    ]

    @functools.partial(
        pl.kernel,
        out_type=jax.ShapeDtypeStruct((T, D, B), jnp.float32),
        mesh=mesh,
        scratch_types=scratch,
        compiler_params=pltpu.CompilerParams(use_tc_tiling_on_sc=True, needs_layout_passes=False),
    )
    def k(xp_hbm, xh_hbm, we2_hbm, pe_hbm, out_hbm,
          pe_v, xp_r, xh_r, rows_r, obuf_r, sem_ip, sem_ih, sem_g, sem_o):
        wid = lax.axis_index("s") * NC + lax.axis_index("c")
        b0 = wid * C
        pltpu.sync_copy(pe_hbm, pe_v)

        def idx_cp(g, b, kind):
            src = (xp_hbm, xh_hbm)[kind]
            dst = (xp_r, xh_r)[kind]
            sem = (sem_ip, sem_ih)[kind]
            return pltpu.make_async_copy(
                src.at[g, pl.ds(b0, C)], dst.at[b], sem[b]
            )

        def gather(g, b):
            return pltpu.make_async_copy(
                we2_hbm.at[xp_r.at[b]], rows_r.at[b], sem_g[b]
            )

        def writeout(g, b):
            return pltpu.make_async_copy(
                obuf_r.at[b], out_hbm.at[g, :, pl.ds(b0, C)], sem_o[b]
            )

        for g in range(3):
            idx_cp(g, g, 0).start()
            idx_cp(g, g, 1).start()
        for g in range(2):
            idx_cp(g, g, 0).wait()
            gather(g, g).start()

        iota = lax.iota(jnp.int32, _L)

        def outer(gg, carry):
            for b in range(_NB):
                g = gg * _NB + b

                @pl.when(g + 3 < T)
                def _():
                    idx_cp(g + 3, (b + 3) % _NB, 0).start()
                    idx_cp(g + 3, (b + 3) % _NB, 1).start()

                @pl.when(g + 2 < T)
                def _():
                    idx_cp(g + 2, (b + 2) % _NB, 0).wait()
                    gather(g + 2, (b + 2) % _NB).start()

                gather(g, b).wait()
                idx_cp(g, b, 1).wait()

                @pl.when(g >= _NB)
                def _():
                    writeout(g - _NB, b).wait()

                rows2d = rows_r.at[b]
                rowv = []
                colv = []
                for r0 in range(NR):
                    hv = xh_r[b, pl.ds(r0 * _L, _L)]
                    rowv.append(iota + (r0 * _L))
                    colv.append(hv * D)
                peJ = jnp.full((_L,), g * D, jnp.int32)

                def issue(j):
                    vals = [
                        plsc.load_gather(rows2d, [rowv[r0], colv[r0] + j])
                        for r0 in range(NR)
                    ]
                    vals.append(plsc.load_gather(pe_v, [peJ + j]))
                    return tuple(vals)

                def jloop(j, carry, b=b):
                    nxt = issue(jnp.minimum(j + 1, D - 1))
                    pe_s = carry[NR]
                    for r0 in range(NR):
                        obuf_r[b, j, pl.ds(r0 * _L, _L)] = (
                            carry[r0] * scale + pe_s
                        )
                    return nxt

                lax.fori_loop(0, D, jloop, issue(0), unroll=4)
                writeout(g, b).start()
            return carry

        lax.fori_loop(0, T // _NB, outer, 0)
        for g in range(T - _NB, T):
            writeout(g, g % _NB).wait()

    return k


@jax.jit
def kernel(x, We):
    B, T = x.shape
    V, D = We.shape
    scale = math.sqrt(D)
    pe1d = _pos_encoding(T, D, jnp.float32).reshape(T * D)
    xT = x.T
    xp = xT >> 1
    xh = xT & 1
    We2 = We.reshape(V // 2, 2 * D)
    sc = _make_sc_kernel(B, T, V, D, scale)
    out_t = sc(xp, xh, We2, pe1d)
    return out_t.transpose(2, 0, 1)


# speedup vs baseline: 1.2766x; 1.0038x over previous
import functools
import math

import jax
import jax.numpy as jnp
from jax import lax
from jax.experimental import pallas as pl
from jax.experimental.pallas import tpu as pltpu
from jax.experimental.pallas import tpu_sc as plsc

_L = 16
_NB = 4


def _pos_encoding(T_len, d_model, dtype):
    positions = jnp.arange(T_len, dtype=dtype)[:, None]
    i = jnp.arange(0, d_model, 2, dtype=dtype)
    denominator = jnp.exp(i / d_model * math.log(10000.0))
    pe = jnp.zeros((T_len, d_model), dtype=dtype)
    pe = pe.at[:, 0::2].set(jnp.sin(positions / denominator))
    pe = pe.at[:, 1::2].set(jnp.cos(positions / denominator))
    return pe


def _make_sc_kernel(B, T, V, D, scale):
    try:
        info = plsc.get_sparse_core_info()
        NC, NS = info.num_cores, info.num_subcores
    except ValueError:
        NC, NS = 2, 16
    NW = NC * NS
    assert B % NW == 0 and T % _NB == 0 and D % _L == 0
    C = B // NW
    assert C % _L == 0
    NR = C // _L
    mesh = plsc.VectorSubcoreMesh(
        core_axis_name="c",
        subcore_axis_name="s",
        num_cores=NC,
        num_subcores=NS,
    )

    scratch = [
        pltpu.VMEM((T * D,), jnp.float32),
        pltpu.VMEM((_NB, C), jnp.int32),
        pltpu.VMEM((_NB, C), jnp.int32),
        pltpu.VMEM((_NB, C, 2 * D), jnp.float32),
        pltpu.VMEM((_NB, D, C), jnp.float32),
        [pltpu.SemaphoreType.DMA] * _NB,
        [pltpu.SemaphoreType.DMA] * _NB,
        [pltpu.SemaphoreType.DMA] * _NB,
        [pltpu.SemaphoreType.DMA] * _NB,
    ]

    @functools.partial(
        pl.kernel,
        out_type=jax.ShapeDtypeStruct((T, D, B), jnp.float32),
        mesh=mesh,
        scratch_types=scratch,
        compiler_params=pltpu.CompilerParams(use_tc_tiling_on_sc=True, needs_layout_passes=False),
    )
    def k(xp_hbm, xh_hbm, we2_hbm, pe_hbm, out_hbm,
          pe_v, xp_r, xh_r, rows_r, obuf_r, sem_ip, sem_ih, sem_g, sem_o):
        wid = lax.axis_index("s") * NC + lax.axis_index("c")
        b0 = wid * C
        pltpu.sync_copy(pe_hbm, pe_v)

        def idx_cp(g, b, kind):
            src = (xp_hbm, xh_hbm)[kind]
            dst = (xp_r, xh_r)[kind]
            sem = (sem_ip, sem_ih)[kind]
            return pltpu.make_async_copy(
                src.at[g, pl.ds(b0, C)], dst.at[b], sem[b]
            )

        def gather(g, b):
            return pltpu.make_async_copy(
                we2_hbm.at[xp_r.at[b]], rows_r.at[b], sem_g[b]
            )

        def writeout(g, b):
            return pltpu.make_async_copy(
                obuf_r.at[b], out_hbm.at[g, :, pl.ds(b0, C)], sem_o[b]
            )

        for g in range(_NB):
            idx_cp(g, g, 0).start()
            idx_cp(g, g, 1).start()
        for g in range(3):
            idx_cp(g, g, 0).wait()
            gather(g, g).start()

        iota = lax.iota(jnp.int32, _L)

        def outer(gg, carry):
            for b in range(_NB):
                g = gg * _NB + b

                gather(g, b).wait()

                @pl.when(g + _NB < T)
                def _():
                    idx_cp(g + _NB, b, 0).start()

                @pl.when(g + 3 < T)
                def _():
                    idx_cp(g + 3, (b + 3) % _NB, 0).wait()
                    gather(g + 3, (b + 3) % _NB).start()

                idx_cp(g, b, 1).wait()

                @pl.when(g >= _NB)
                def _():
                    writeout(g - _NB, b).wait()

                rows2d = rows_r.at[b]
                rowv = []
                colv = []
                for r0 in range(NR):
                    hv = xh_r[b, pl.ds(r0 * _L, _L)]
                    rowv.append(iota + (r0 * _L))
                    colv.append(hv * D)
                peJ = jnp.full((_L,), g * D, jnp.int32)

                def issue(j):
                    vals = [
                        plsc.load_gather(rows2d, [rowv[r0], colv[r0] + j])
                        for r0 in range(NR)
                    ]
                    vals.append(plsc.load_gather(pe_v, [peJ + j]))
                    return tuple(vals)

                def jloop(j, carry, b=b):
                    nxt = issue(jnp.minimum(j + 1, D - 1))
                    pe_s = carry[NR]
                    for r0 in range(NR):
                        obuf_r[b, j, pl.ds(r0 * _L, _L)] = (
                            carry[r0] * scale + pe_s
                        )
                    return nxt

                lax.fori_loop(0, D, jloop, issue(0), unroll=4)

                @pl.when(g + _NB < T)
                def _():
                    idx_cp(g + _NB, b, 1).start()

                writeout(g, b).start()
            return carry

        lax.fori_loop(0, T // _NB, outer, 0)
        for g in range(T - _NB, T):
            writeout(g, g % _NB).wait()

    return k


@jax.jit
def kernel(x, We):
    B, T = x.shape
    V, D = We.shape
    scale = math.sqrt(D)
    pe1d = _pos_encoding(T, D, jnp.float32).reshape(T * D)
    xT = x.T
    xp = xT >> 1
    xh = xT & 1
    We2 = We.reshape(V // 2, 2 * D)
    sc = _make_sc_kernel(B, T, V, D, scale)
    out_t = sc(xp, xh, We2, pe1d)
    return out_t.transpose(2, 0, 1)


# speedup vs baseline: 2.4927x; 1.9527x over previous
import functools
import math

import jax
import jax.numpy as jnp
from jax import lax
from jax.experimental import pallas as pl
from jax.experimental.pallas import tpu as pltpu
from jax.experimental.pallas import tpu_sc as plsc

_L = 16
_NB = 4


def _pos_encoding(T_len, d_model, dtype):
    positions = jnp.arange(T_len, dtype=dtype)[:, None]
    i = jnp.arange(0, d_model, 2, dtype=dtype)
    denominator = jnp.exp(i / d_model * math.log(10000.0))
    pe = jnp.zeros((T_len, d_model), dtype=dtype)
    pe = pe.at[:, 0::2].set(jnp.sin(positions / denominator))
    pe = pe.at[:, 1::2].set(jnp.cos(positions / denominator))
    return pe


def _make_sc_kernel(B, T, V, D, scale):
    try:
        info = plsc.get_sparse_core_info()
        NC, NS = info.num_cores, info.num_subcores
    except ValueError:
        NC, NS = 2, 16
    NW = NC * NS
    assert B % NW == 0 and T % _NB == 0 and D % _L == 0
    C = B // NW
    assert C % _L == 0
    NR = C // _L
    mesh = plsc.VectorSubcoreMesh(
        core_axis_name="c",
        subcore_axis_name="s",
        num_cores=NC,
        num_subcores=NS,
    )

    scratch = [
        pltpu.VMEM((T * D,), jnp.float32),
        pltpu.VMEM((_NB, C), jnp.int32),
        pltpu.VMEM((_NB, C), jnp.int32),
        pltpu.VMEM((_NB, C, 2 * D), jnp.float32),
        pltpu.VMEM((_NB, D, C), jnp.float32),
        [pltpu.SemaphoreType.DMA] * _NB,
        [pltpu.SemaphoreType.DMA] * _NB,
        [pltpu.SemaphoreType.DMA] * _NB,
        [pltpu.SemaphoreType.DMA] * _NB,
    ]

    @functools.partial(
        pl.kernel,
        out_type=jax.ShapeDtypeStruct((T, D, B), jnp.float32),
        mesh=mesh,
        scratch_types=scratch,
        compiler_params=pltpu.CompilerParams(use_tc_tiling_on_sc=True, needs_layout_passes=False),
    )
    def k(xp_hbm, xh_hbm, we2_hbm, pe_hbm, out_hbm,
          pe_v, xp_r, xh_r, rows_r, obuf_r, sem_ip, sem_ih, sem_g, sem_o):
        wid = lax.axis_index("s") * NC + lax.axis_index("c")
        b0 = wid * C
        pltpu.sync_copy(pe_hbm, pe_v)

        def idx_cp(g, b, kind):
            src = (xp_hbm, xh_hbm)[kind]
            dst = (xp_r, xh_r)[kind]
            sem = (sem_ip, sem_ih)[kind]
            return pltpu.make_async_copy(
                src.at[g, pl.ds(b0, C)], dst.at[b], sem[b]
            )

        def gather(g, b):
            return pltpu.make_async_copy(
                we2_hbm.at[xp_r.at[b]], rows_r.at[b], sem_g[b]
            )

        def writeout(g, b):
            return pltpu.make_async_copy(
                obuf_r.at[b], out_hbm.at[g, :, pl.ds(b0, C)], sem_o[b]
            )

        for g in range(_NB):
            idx_cp(g, g, 0).start()
            idx_cp(g, g, 1).start()
        for g in range(3):
            idx_cp(g, g, 0).wait()
            gather(g, g).start()

        iota = lax.iota(jnp.int32, _L)

        def outer(gg, carry):
            for b in range(_NB):
                g = gg * _NB + b

                gather(g, b).wait()

                @pl.when(g + _NB < T)
                def _():
                    idx_cp(g + _NB, b, 0).start()

                @pl.when(g + 3 < T)
                def _():
                    idx_cp(g + 3, (b + 3) % _NB, 0).wait()
                    gather(g + 3, (b + 3) % _NB).start()

                idx_cp(g, b, 1).wait()

                @pl.when(g >= _NB)
                def _():
                    writeout(g - _NB, b).wait()

                rows2d = rows_r.at[b]
                rowv = []
                colv = []
                for r0 in range(NR):
                    hv = xh_r[b, pl.ds(r0 * _L, _L)]
                    rowv.append(iota + (r0 * _L))
                    colv.append(hv * D)
                peJ = jnp.full((_L,), g * D, jnp.int32)

                def issue(j):
                    jv = (j + iota) & (D - 1)
                    vals = [
                        plsc.load_gather(rows2d, [rowv[r0], colv[r0] + jv])
                        for r0 in range(NR)
                    ]
                    vals.append(plsc.load_gather(pe_v, [peJ + jv]))
                    vals.append(jv)
                    return tuple(vals)

                obuf2d = obuf_r.at[b]

                def jloop(j, carry, obuf2d=obuf2d):
                    nxt = issue(jnp.minimum(j + 1, D - 1))
                    pe_s = carry[NR]
                    jv = carry[NR + 1]
                    for r0 in range(NR):
                        plsc.store_scatter(
                            obuf2d, [jv, rowv[r0]], carry[r0] * scale + pe_s
                        )
                    return nxt

                lax.fori_loop(0, D, jloop, issue(0), unroll=4)

                @pl.when(g + _NB < T)
                def _():
                    idx_cp(g + _NB, b, 1).start()

                writeout(g, b).start()
            return carry

        lax.fori_loop(0, T // _NB, outer, 0)
        for g in range(T - _NB, T):
            writeout(g, g % _NB).wait()

    return k


@jax.jit
def kernel(x, We):
    B, T = x.shape
    V, D = We.shape
    scale = math.sqrt(D)
    pe1d = _pos_encoding(T, D, jnp.float32).reshape(T * D)
    xT = x.T
    xp = xT >> 1
    xh = xT & 1
    We2 = We.reshape(V // 2, 2 * D)
    sc = _make_sc_kernel(B, T, V, D, scale)
    out_t = sc(xp, xh, We2, pe1d)
    return out_t.transpose(2, 0, 1)
